# Initial kernel scaffold; baseline (speedup 1.0000x reference)
import jax
import jax.numpy as jnp
from jax.experimental import pallas as pl


def kernel(cos_pred, mask_gt):
    raise NotImplementedError("write your pallas kernel here")


# speedup vs baseline: 1.0317x; 1.0317x over previous
import jax
import jax.numpy as jnp
from jax.experimental import pallas as pl
from jax.experimental.pallas import tpu as pltpu

_MARGIN = 0.1
_N = 4096
_BM = 256
_NBLK = _N // _BM


def _loss_body(cos_ref, mask_ref, out_ref, acc_ref):
    i = pl.program_id(0)

    @pl.when(i == 0)
    def _init():
        acc_ref[0] = 0.0
        acc_ref[1] = 0.0

    c = cos_ref[...]
    m = mask_ref[...].astype(jnp.float32)
    cnt_t = jnp.sum(m, axis=1, keepdims=True)
    cnt_nt = _N - cnt_t
    lt_num = jnp.sum((1.0 - c) * m, axis=1, keepdims=True)
    r = jnp.maximum(c - _MARGIN, 0.0)
    lnt_num = jnp.sum(r, axis=1, keepdims=True) - jnp.sum(r * m, axis=1, keepdims=True)
    lt = lt_num / cnt_t
    lnt = lnt_num / cnt_nt
    acc_ref[0] += jnp.sum(lt)
    acc_ref[1] += jnp.sum(lnt)

    @pl.when(i == _NBLK - 1)
    def _emit():
        lt_mean = acc_ref[0] / _N
        lnt_mean = acc_ref[1] / _N
        out_ref[0] = (lt_mean + lnt_mean) * 0.5
        out_ref[1] = lt_mean
        out_ref[2] = lnt_mean


def kernel(cos_pred, mask_gt):
    out = pl.pallas_call(
        _loss_body,
        grid=(_NBLK,),
        in_specs=[
            pl.BlockSpec((_BM, _N), lambda i: (i, 0)),
            pl.BlockSpec((_BM, _N), lambda i: (i, 0)),
        ],
        out_specs=pl.BlockSpec(memory_space=pltpu.SMEM),
        out_shape=jax.ShapeDtypeStruct((3,), jnp.float32),
        scratch_shapes=[pltpu.SMEM((2,), jnp.float32)],
    )(cos_pred, mask_gt)
    return (out[0], out[1], out[2])
